# baseline (speedup 1.0000x reference)
import jax
import jax.numpy as jnp
from jax import lax
from jax.experimental import pallas as pl
from jax.experimental.pallas import tpu as pltpu
from jax.experimental.pallas import tpu_sc as plsc

NC = 2
NS = 16
NW = NC * NS
B = 16384
SEQ = 50
D = 32

ROWS_W = B // NW
NB = 64
NCHUNK = ROWS_W // NB
T = NB * SEQ
NCHAIN = 4


def _body(tok_hbm, tab_hbm, out_hbm,
          tok_a, rows_a, tok_b, rows_b, sums_v, inv_v, w0_v, row0_v,
          stab, stg_f, stg_b, sem_a, sem_b):
    cid = lax.axis_index("c")
    sid = lax.axis_index("s")
    wid = sid * NC + cid
    base_row = wid * ROWS_W

    pltpu.sync_copy(tab_hbm.at[pl.ds(0, 1)], row0_v)

    def stage_tab(s, carry):
        r0 = sid * 625 + s * 125
        pltpu.sync_copy(tab_hbm.at[pl.ds(r0, 125)], stg_f)

        def cvt(r, carry):
            stg_b[r, pl.ds(0, 32)] = plsc.pack(
                stg_f[r, pl.ds(0, 16)], stg_f[r, pl.ds(16, 16)],
                format=plsc.PackFormat.INTERLEAVED)
            return carry

        lax.fori_loop(0, 125, cvt, 0)
        pltpu.sync_copy(stg_b, stab.at[pl.ds(r0, 125)])
        return carry

    lax.fori_loop(0, 5, stage_tab, 0)
    plsc.subcore_barrier()

    def stage(c, tok_v, rows_v, sem):
        row0 = base_row + c * NB
        pltpu.sync_copy(tok_hbm.at[pl.ds(row0 * SEQ, T)], tok_v)
        pltpu.async_copy(stab.at[tok_v], rows_v, sem)

    def finish(c, tok_v, rows_v, sem):
        def count_grp(g, carry):
            lanes50 = lax.iota(jnp.int32, 16) * SEQ + g * (16 * SEQ)

            def cl(l, cnt):
                t = plsc.load_gather(tok_v, [lanes50 + l])
                return cnt + jnp.where(t != 0, 1, 0)

            cnt = lax.fori_loop(0, SEQ, cl, jnp.zeros((16,), jnp.int32))
            cntf = cnt.astype(jnp.float32)
            inv = 1.0 / jnp.maximum(cntf, 1.0)
            inv_v[pl.ds(g * 16, 16)] = inv
            w0_v[pl.ds(g * 16, 16)] = (float(SEQ) - cntf) * inv
            return carry

        lax.fori_loop(0, NB // 16, count_grp, 0)

        r0e = row0_v[0, pl.ds(0, 16)]
        r0o = row0_v[0, pl.ds(16, 16)]

        pltpu.make_async_copy(stab.at[tok_v], rows_v, sem).wait()

        def acc_body(i, carry):
            r0 = i * SEQ
            si = plsc.load_gather(inv_v, [jnp.full((16,), i, jnp.int32)])
            sw = plsc.load_gather(w0_v, [jnp.full((16,), i, jnp.int32)])

            parts = []
            for g in range(SEQ // 4):
                b = r0 + 4 * g
                s01 = rows_v[b, pl.ds(0, 32)] + rows_v[b + 1, pl.ds(0, 32)]
                s23 = rows_v[b + 2, pl.ds(0, 32)] + rows_v[b + 3, pl.ds(0, 32)]
                parts.append(s01 + s23)
            parts.append(
                rows_v[r0 + 48, pl.ds(0, 32)] + rows_v[r0 + 49, pl.ds(0, 32)])
            ev, od = [], []
            for k, p in enumerate(parts):
                e, o = plsc.unpack(p, format=plsc.PackFormat.INTERLEAVED)
                if k < NCHAIN:
                    ev.append(e)
                    od.append(o)
                else:
                    ev[k % NCHAIN] = ev[k % NCHAIN] + e
                    od[k % NCHAIN] = od[k % NCHAIN] + o
            a0 = (ev[0] + ev[1]) + (ev[2] + ev[3])
            a1 = (od[0] + od[1]) + (od[2] + od[3])
            sums_v[i, pl.ds(0, 16)] = a0 * si - sw * r0e
            sums_v[i, pl.ds(16, 16)] = a1 * si - sw * r0o
            return carry

        lax.fori_loop(0, NB, acc_body, 0)
        pltpu.sync_copy(sums_v, out_hbm.at[pl.ds(base_row + c * NB, NB)])

    stage(0, tok_a, rows_a, sem_a)

    def pair(p, carry):
        c = 2 * p
        stage(c + 1, tok_b, rows_b, sem_b)
        finish(c, tok_a, rows_a, sem_a)

        @pl.when(c + 2 < NCHUNK)
        def _():
            stage(c + 2, tok_a, rows_a, sem_a)

        finish(c + 1, tok_b, rows_b, sem_b)
        return carry

    lax.fori_loop(0, NCHUNK // 2, pair, 0)


@jax.jit
def _run(tok, tab):
    mesh = plsc.VectorSubcoreMesh(core_axis_name="c", subcore_axis_name="s")
    return pl.kernel(
        _body,
        out_type=jax.ShapeDtypeStruct((B, D), jnp.float32),
        mesh=mesh,
        compiler_params=pltpu.CompilerParams(
            use_tc_tiling_on_sc=False, needs_layout_passes=False),
        scratch_types=[
            pltpu.VMEM((T,), jnp.int32),
            pltpu.VMEM((T, D), jnp.bfloat16),
            pltpu.VMEM((T,), jnp.int32),
            pltpu.VMEM((T, D), jnp.bfloat16),
            pltpu.VMEM((NB, D), jnp.float32),
            pltpu.VMEM((NB,), jnp.float32),
            pltpu.VMEM((NB,), jnp.float32),
            pltpu.VMEM((1, D), jnp.float32),
            pltpu.VMEM_SHARED((10000, D), jnp.bfloat16),
            pltpu.VMEM((125, D), jnp.float32),
            pltpu.VMEM((125, D), jnp.bfloat16),
            pltpu.SemaphoreType.DMA,
            pltpu.SemaphoreType.DMA,
        ],
    )(tok, tab)


def kernel(token_ids, table):
    return _run(token_ids.reshape(-1), table)


# speedup vs baseline: 1.0694x; 1.0694x over previous
import jax
import jax.numpy as jnp
from jax import lax
from jax.experimental import pallas as pl
from jax.experimental.pallas import tpu as pltpu
from jax.experimental.pallas import tpu_sc as plsc

NC = 2
NS = 16
NW = NC * NS
B = 16384
SEQ = 50
D = 32

ROWS_W = B // NW
NB = 64
NCHUNK = ROWS_W // NB
T = NB * SEQ
NCHAIN = 4


def _body(tok_hbm, tab_hbm, out_hbm,
          tok_a, rows_a, tok_b, rows_b, sums_v, inv_v, w0_v, row0_v,
          stab, sem_a, sem_b):
    cid = lax.axis_index("c")
    sid = lax.axis_index("s")
    wid = sid * NC + cid
    base_row = wid * ROWS_W

    pltpu.sync_copy(tab_hbm.at[pl.ds(0, 1)], row0_v)

    pltpu.sync_copy(tab_hbm.at[pl.ds(sid * 625, 625)],
                    stab.at[pl.ds(sid * 625, 625)])
    plsc.subcore_barrier()

    def stage(c, tok_v, rows_v, sem):
        row0 = base_row + c * NB
        pltpu.sync_copy(tok_hbm.at[pl.ds(row0 * SEQ, T)], tok_v)
        pltpu.async_copy(stab.at[tok_v], rows_v, sem)

    def finish(c, tok_v, rows_v, sem):
        def count_grp(g, carry):
            lanes50 = lax.iota(jnp.int32, 16) * SEQ + g * (16 * SEQ)

            def cl(l, cnt):
                t = plsc.load_gather(tok_v, [lanes50 + l])
                return cnt + jnp.where(t != 0, 1, 0)

            cnt = lax.fori_loop(0, SEQ, cl, jnp.zeros((16,), jnp.int32))
            cntf = cnt.astype(jnp.float32)
            inv = 1.0 / jnp.maximum(cntf, 1.0)
            inv_v[pl.ds(g * 16, 16)] = inv
            w0_v[pl.ds(g * 16, 16)] = (float(SEQ) - cntf) * inv
            return carry

        lax.fori_loop(0, NB // 16, count_grp, 0)

        r0e, r0o = plsc.unpack(
            row0_v[0, pl.ds(0, 32)], format=plsc.PackFormat.INTERLEAVED)
        lane = lax.iota(jnp.int32, 16)
        half = lane >> 1
        is_ev = (lane & 1) == 0

        def _splat16(x, idx):
            return lax.gather(
                x, idx.reshape(16, 1),
                lax.GatherDimensionNumbers(
                    offset_dims=(), collapsed_slice_dims=(0,),
                    start_index_map=(0,)),
                (1,),
                mode=lax.GatherScatterMode.PROMISE_IN_BOUNDS)

        pltpu.make_async_copy(stab.at[tok_v], rows_v, sem).wait()

        def acc_body(i, carry):
            r0 = i * SEQ
            si = plsc.load_gather(inv_v, [jnp.full((16,), i, jnp.int32)])
            sw = plsc.load_gather(w0_v, [jnp.full((16,), i, jnp.int32)])

            parts = []
            for g in range(SEQ // 4):
                b = r0 + 4 * g
                s01 = rows_v[b, pl.ds(0, 32)] + rows_v[b + 1, pl.ds(0, 32)]
                s23 = rows_v[b + 2, pl.ds(0, 32)] + rows_v[b + 3, pl.ds(0, 32)]
                parts.append(s01 + s23)
            parts.append(
                rows_v[r0 + 48, pl.ds(0, 32)] + rows_v[r0 + 49, pl.ds(0, 32)])
            ev, od = [], []
            for k, p in enumerate(parts):
                e, o = plsc.unpack(p, format=plsc.PackFormat.INTERLEAVED)
                if k < NCHAIN:
                    ev.append(e)
                    od.append(o)
                else:
                    ev[k % NCHAIN] = ev[k % NCHAIN] + e
                    od[k % NCHAIN] = od[k % NCHAIN] + o
            a0 = (ev[0] + ev[1]) + (ev[2] + ev[3])
            a1 = (od[0] + od[1]) + (od[2] + od[3])
            a0 = a0 * si - sw * r0e
            a1 = a1 * si - sw * r0o
            lo = jnp.where(is_ev, _splat16(a0, half), _splat16(a1, half))
            hi = jnp.where(is_ev, _splat16(a0, half + 8), _splat16(a1, half + 8))
            sums_v[i, pl.ds(0, 16)] = lo
            sums_v[i, pl.ds(16, 16)] = hi
            return carry

        lax.fori_loop(0, NB, acc_body, 0)
        pltpu.sync_copy(sums_v, out_hbm.at[pl.ds(base_row + c * NB, NB)])

    stage(0, tok_a, rows_a, sem_a)

    def pair(p, carry):
        c = 2 * p
        stage(c + 1, tok_b, rows_b, sem_b)
        finish(c, tok_a, rows_a, sem_a)

        @pl.when(c + 2 < NCHUNK)
        def _():
            stage(c + 2, tok_a, rows_a, sem_a)

        finish(c + 1, tok_b, rows_b, sem_b)
        return carry

    lax.fori_loop(0, NCHUNK // 2, pair, 0)


@jax.jit
def _run(tok, tab):
    mesh = plsc.VectorSubcoreMesh(core_axis_name="c", subcore_axis_name="s")
    return pl.kernel(
        _body,
        out_type=jax.ShapeDtypeStruct((B, D), jnp.float32),
        mesh=mesh,
        compiler_params=pltpu.CompilerParams(
            use_tc_tiling_on_sc=False, needs_layout_passes=False),
        scratch_types=[
            pltpu.VMEM((T,), jnp.int32),
            pltpu.VMEM((T, D), jnp.bfloat16),
            pltpu.VMEM((T,), jnp.int32),
            pltpu.VMEM((T, D), jnp.bfloat16),
            pltpu.VMEM((NB, D), jnp.float32),
            pltpu.VMEM((NB,), jnp.float32),
            pltpu.VMEM((NB,), jnp.float32),
            pltpu.VMEM((1, D), jnp.bfloat16),
            pltpu.VMEM_SHARED((10000, D), jnp.bfloat16),
            pltpu.SemaphoreType.DMA,
            pltpu.SemaphoreType.DMA,
        ],
    )(tok, tab)


def kernel(token_ids, table):
    return _run(token_ids.reshape(-1), table.astype(jnp.bfloat16))
